# baseline (speedup 1.0000x reference)
import jax
import jax.numpy as jnp
from jax.experimental import pallas as pl
from jax.experimental.pallas import tpu as pltpu

_N = 32768
_D = 768
_H = 64
_T = 64
_O = 64
_BLK = 2048


def _half(x, ti, W1_ref, b1_ref, W2_ref, b2_ref, W3_ref, b3_ref,
          W2d_ref, hb_ref):
    _HB = x.shape[0]
    u = (jnp.dot(x[:, :_D // 2], W1_ref[:_D // 2, :],
                 preferred_element_type=jnp.float32)
         + jnp.dot(x[:, _D // 2:], W1_ref[_D // 2:, :],
                   preferred_element_type=jnp.float32))
    h = jnp.tanh(u + b1_ref[...]).astype(jnp.bfloat16)
    h = jnp.tanh(jnp.dot(h, W2_ref[...], preferred_element_type=jnp.float32)
                 + b2_ref[...]).astype(jnp.bfloat16)
    h = jnp.tanh(jnp.dot(h, W3_ref[...], preferred_element_type=jnp.float32)
                 + b3_ref[...]).astype(jnp.bfloat16)
    grp = jax.lax.broadcasted_iota(jnp.int32, (_HB, 8 * _H), 1) // _H
    h8 = jnp.concatenate([h] * 8, axis=1)
    hb8 = jnp.where(grp == ti % 8, h8, jnp.bfloat16(0))
    z = jnp.dot(hb8, W2d_ref[...], preferred_element_type=jnp.float32)
    z = jnp.where(grp == ti // 8, z, 0.0)
    w = (8 * _O) // 2
    while w >= _O:
        z = z[:, :w] + z[:, w:]
        w //= 2
    onehot = (jax.lax.broadcasted_iota(jnp.int32, (_HB, _T), 1)
              == ti).astype(jnp.float32)
    return z + jnp.dot(onehot, hb_ref[...],
                       preferred_element_type=jnp.float32)


def _trunk_head_kernel(x_ref, ti_ref, W1_ref, b1_ref, W2_ref, b2_ref,
                       W3_ref, b3_ref, W2d_ref, hb_ref, out_ref):
    m = _BLK // 4
    ti = ti_ref[0, 0, :].reshape(_BLK, 1)
    for s in range(4):
        out_ref[s * m:(s + 1) * m, :] = _half(
            x_ref[s * m:(s + 1) * m, :].astype(jnp.bfloat16),
            ti[s * m:(s + 1) * m],
            W1_ref, b1_ref, W2_ref, b2_ref, W3_ref, b3_ref,
            W2d_ref, hb_ref)


def kernel(inputs, task_indices, W1, b1, W2, b2, W3, b3, headW, headb):
    n_blocks = _N // _BLK
    ti3 = task_indices.astype(jnp.int32).reshape(n_blocks, 1, _BLK)
    W1 = W1.astype(jnp.bfloat16)
    W2 = W2.astype(jnp.bfloat16)
    W3 = W3.astype(jnp.bfloat16)
    W2d = (headW.reshape(8, 8, _H, _O).transpose(1, 2, 0, 3)
           .reshape(8 * _H, 8 * _O).astype(jnp.bfloat16))
    b1r = b1.reshape(1, _H)
    b2r = b2.reshape(1, _H)
    b3r = b3.reshape(1, _H)

    grid = (n_blocks,)
    out = pl.pallas_call(
        _trunk_head_kernel,
        grid=grid,
        in_specs=[
            pl.BlockSpec((_BLK, _D), lambda i: (i, 0)),
            pl.BlockSpec((1, 1, _BLK), lambda i: (i, 0, 0)),
            pl.BlockSpec((_D, _H), lambda i: (0, 0)),
            pl.BlockSpec((1, _H), lambda i: (0, 0)),
            pl.BlockSpec((_H, _H), lambda i: (0, 0)),
            pl.BlockSpec((1, _H), lambda i: (0, 0)),
            pl.BlockSpec((_H, _H), lambda i: (0, 0)),
            pl.BlockSpec((1, _H), lambda i: (0, 0)),
            pl.BlockSpec((8 * _H, 8 * _O), lambda i: (0, 0)),
            pl.BlockSpec((_T, _O), lambda i: (0, 0)),
        ],
        out_specs=pl.BlockSpec((_BLK, _O), lambda i: (i, 0)),
        out_shape=jax.ShapeDtypeStruct((_N, _O), jnp.float32),
        compiler_params=pltpu.CompilerParams(
            dimension_semantics=("parallel",)),
    )(inputs, ti3, W1, b1r, W2, b2r, W3, b3r, W2d, headb)
    return out


# speedup vs baseline: 1.1657x; 1.1657x over previous
import jax
import jax.numpy as jnp
from jax.experimental import pallas as pl
from jax.experimental.pallas import tpu as pltpu

_N = 32768
_D = 768
_H = 64
_T = 64
_O = 64
_BLK = 2048


def _half(x, ti, W1_ref, b1_ref, W2_ref, b2_ref, W3_ref, b3_ref,
          W2d_ref, hb_ref):
    _HB = x.shape[0]
    u = (jnp.dot(x[:, :_D // 2], W1_ref[:_D // 2, :],
                 preferred_element_type=jnp.float32)
         + jnp.dot(x[:, _D // 2:], W1_ref[_D // 2:, :],
                   preferred_element_type=jnp.float32))
    h = jnp.tanh(u + b1_ref[...]).astype(jnp.bfloat16)
    h = jnp.tanh(jnp.dot(h, W2_ref[...], preferred_element_type=jnp.float32)
                 + b2_ref[...]).astype(jnp.bfloat16)
    h = jnp.tanh(jnp.dot(h, W3_ref[...], preferred_element_type=jnp.float32)
                 + b3_ref[...]).astype(jnp.bfloat16)
    grp = jax.lax.broadcasted_iota(jnp.int32, (_HB, 8 * _H), 1) // _H
    h8 = jnp.concatenate([h] * 8, axis=1)
    hb8 = jnp.where(grp == ti % 8, h8, jnp.bfloat16(0))
    z = jnp.dot(hb8, W2d_ref[...], preferred_element_type=jnp.float32)
    z = jnp.where(grp == ti // 8, z, 0.0)
    w = (8 * _O) // 2
    while w >= _O:
        z = z[:, :w] + z[:, w:]
        w //= 2
    onehot = (jax.lax.broadcasted_iota(jnp.int32, (_HB, _T), 1)
              == ti).astype(jnp.float32)
    return z + jnp.dot(onehot, hb_ref[...],
                       preferred_element_type=jnp.float32)


def _trunk_head_kernel(x_ref, ti_ref, W1_ref, b1_ref, W2_ref, b2_ref,
                       W3_ref, b3_ref, W2d_ref, hb_ref, out_ref):
    m = _BLK // 2
    ti = ti_ref[0, 0, :].reshape(_BLK, 1)
    out_ref[:m, :] = _half(x_ref[:m, :].astype(jnp.bfloat16), ti[:m],
                           W1_ref, b1_ref, W2_ref, b2_ref, W3_ref, b3_ref,
                           W2d_ref, hb_ref)
    out_ref[m:, :] = _half(x_ref[m:, :].astype(jnp.bfloat16), ti[m:],
                           W1_ref, b1_ref, W2_ref, b2_ref, W3_ref, b3_ref,
                           W2d_ref, hb_ref)


def kernel(inputs, task_indices, W1, b1, W2, b2, W3, b3, headW, headb):
    n_blocks = _N // _BLK
    ti3 = task_indices.astype(jnp.int32).reshape(n_blocks, 1, _BLK)
    W1 = W1.astype(jnp.bfloat16)
    W2 = W2.astype(jnp.bfloat16)
    W3 = W3.astype(jnp.bfloat16)
    W2d = (headW.reshape(8, 8, _H, _O).transpose(1, 2, 0, 3)
           .reshape(8 * _H, 8 * _O).astype(jnp.bfloat16))
    b1r = b1.reshape(1, _H)
    b2r = b2.reshape(1, _H)
    b3r = b3.reshape(1, _H)

    grid = (n_blocks,)
    out = pl.pallas_call(
        _trunk_head_kernel,
        grid=grid,
        in_specs=[
            pl.BlockSpec((_BLK, _D), lambda i: (i, 0)),
            pl.BlockSpec((1, 1, _BLK), lambda i: (i, 0, 0)),
            pl.BlockSpec((_D, _H), lambda i: (0, 0)),
            pl.BlockSpec((1, _H), lambda i: (0, 0)),
            pl.BlockSpec((_H, _H), lambda i: (0, 0)),
            pl.BlockSpec((1, _H), lambda i: (0, 0)),
            pl.BlockSpec((_H, _H), lambda i: (0, 0)),
            pl.BlockSpec((1, _H), lambda i: (0, 0)),
            pl.BlockSpec((8 * _H, 8 * _O), lambda i: (0, 0)),
            pl.BlockSpec((_T, _O), lambda i: (0, 0)),
        ],
        out_specs=pl.BlockSpec((_BLK, _O), lambda i: (i, 0)),
        out_shape=jax.ShapeDtypeStruct((_N, _O), jnp.float32),
        compiler_params=pltpu.CompilerParams(
            dimension_semantics=("parallel",)),
    )(inputs, ti3, W1, b1r, W2, b2r, W3, b3r, W2d, headb)
    return out
